# baseline (speedup 1.0000x reference)
import functools

import jax
import jax.numpy as jnp
from jax import lax
from jax.experimental import pallas as pl
from jax.experimental.pallas import tpu as pltpu
from jax.experimental.pallas import tpu_sc as plsc


def kernel(user_id, timestamp, user_table, time_table, buckets):
    B = user_id.shape[0]
    V = user_table.shape[0]
    UD = user_table.shape[1]
    TD = time_table.shape[1]
    NB = buckets.shape[0]
    W = UD + TD

    ut = user_table.T
    NBLK = V // W
    TAILLO = NBLK * W
    NTAIL = V - TAILLO
    utail = jnp.pad(user_table[TAILLO:], ((0, 0), (0, W - UD)))
    ttz = jnp.pad(time_table, ((0, 0), (UD, 0)))

    info = plsc.get_sparse_core_info()
    NC, NS, L = info.num_cores, info.num_subcores, info.num_lanes
    NW = NC * NS
    bpw = B // NW
    nchunks = bpw // L
    nuvr = B // L
    CAP = 656

    mesh = plsc.VectorSubcoreMesh(core_axis_name="c", subcore_axis_name="s")

    @functools.partial(
        pl.kernel,
        out_type=(
            jax.ShapeDtypeStruct((B + 8, W), jnp.float32),
            jax.ShapeDtypeStruct((B, W), jnp.float32),
        ),
        mesh=mesh,
        compiler_params=pltpu.CompilerParams(needs_layout_passes=False),
        scratch_types=[
            pltpu.VMEM((B,), jnp.int32),
            pltpu.VMEM((bpw,), jnp.float32),
            pltpu.VMEM((NB,), jnp.float32),
            pltpu.VMEM((bpw,), jnp.int32),
            pltpu.VMEM((CAP, W), jnp.float32),
            pltpu.VMEM((CAP,), jnp.int32),
            pltpu.VMEM((CAP,), jnp.int32),
            pltpu.VMEM((CAP,), jnp.int32),
            pltpu.VMEM((704,), jnp.int32),
            pltpu.VMEM((704,), jnp.int32),
            pltpu.VMEM((64, W), jnp.float32),
            pltpu.VMEM((64, W), jnp.float32),
            pltpu.VMEM((3 * L,), jnp.int32),
            pltpu.VMEM((3 * L,), jnp.int32),
            pltpu.VMEM((3 * L, W), jnp.float32),
            pltpu.SemaphoreType.DMA,
            pltpu.SemaphoreType.DMA,
            pltpu.SemaphoreType.DMA,
            pltpu.SemaphoreType.DMA,
            pltpu.SemaphoreType.DMA,
            pltpu.SemaphoreType.DMA,
        ],
    )
    def body(uid_hbm, ts_hbm, ut_hbm, utail_hbm, ttz_hbm, bkt_hbm,
             outu_hbm, outt_hbm,
             uid_v, ts_v, bkt_v, tidx_v, stag_v, ul_v, bl_v, bg_v,
             lj_v, bj_v, blka_v, blkb_v, tl_v, tb_v, trow_v,
             sem_uid, sem_t, sem_a, sem_b, sem_sc, sem_tl):
        wid = lax.axis_index("s") * NC + lax.axis_index("c")
        base = wid * bpw
        iota = lax.iota(jnp.int32, L)

        ucp = pltpu.async_copy(uid_hbm, uid_v, sem_uid)
        pltpu.sync_copy(bkt_hbm, bkt_v)
        pltpu.sync_copy(ts_hbm.at[pl.ds(base, bpw)], ts_v)

        def chunk(c, carry):
            v = ts_v[pl.ds(c * L, L)]
            ans = jnp.zeros((L,), jnp.int32)
            k = NB
            while k >= 1:
                probe = jnp.minimum(ans + (k - 1), NB - 1)
                bv = plsc.load_gather(bkt_v, [probe])
                pred = (bv <= v) & (ans + k <= NB)
                ans = jnp.where(pred, ans + k, ans)
                k //= 2
            tidx_v[pl.ds(c * L, L)] = ans
            return carry

        lax.fori_loop(0, nchunks, chunk, 0)

        tcp = pltpu.async_copy(ttz_hbm.at[tidx_v], stag_v.at[pl.ds(0, bpw)],
                               sem_t)
        tcp.wait()
        pltpu.sync_copy(stag_v.at[pl.ds(0, bpw)],
                        outt_hbm.at[pl.ds(base, bpw), :])

        zero16 = jnp.zeros((L,), jnp.float32)

        def zrow(r, carry):
            for g in range(UD // L):
                stag_v[r, pl.ds(UD + g * L, L)] = zero16
            return carry

        lax.fori_loop(0, CAP, zrow, 0)

        dumpv = jnp.full((L,), B, jnp.int32)

        def zidx(r, carry):
            bg_v[pl.ds(r * L, L)] = dumpv
            return carry

        lax.fori_loop(0, CAP // L, zidx, 0)
        for g in range(3):
            tb_v[pl.ds(g * L, L)] = dumpv
            tl_v[pl.ds(g * L, L)] = jnp.zeros((L,), jnp.int32)

        ucp.wait()

        def tailscan2(i, off):
            u = uid_v[pl.ds(base + i * L, L)]
            m = u >= TAILLO
            plsc.store_compressed(tl_v.at[pl.ds(off, L)], u - TAILLO, mask=m)
            plsc.store_compressed(tb_v.at[pl.ds(off, L)],
                                  base + i * L + iota, mask=m)
            return off + jnp.max(plsc.all_reduce_population_count(m))

        lax.fori_loop(0, nchunks, tailscan2, 0)
        tlcp = pltpu.async_copy(utail_hbm.at[tl_v], trow_v, sem_tl)
        tlcp.wait()
        ocp_tail = pltpu.async_copy(trow_v, outu_hbm.at[tb_v], sem_tl)
        ocp_tail.wait()

        lo = (wid * NBLK) // NW
        hi = ((wid + 1) * NBLK) // NW

        def scan(i, off):
            u = uid_v[pl.ds(i * L, L)]
            j = lax.shift_right_logical(u, 7)
            m = (j >= lo) & (j < hi)
            plsc.store_compressed(ul_v.at[pl.ds(off, L)], u, mask=m)
            plsc.store_compressed(bl_v.at[pl.ds(off, L)], i * L + iota, mask=m)
            return off + jnp.max(plsc.all_reduce_population_count(m))

        nmatch = lax.fori_loop(0, nuvr, scan, 0)
        nmvr = (nmatch + (L - 1)) // L

        def fire(j, buf, sem):
            return pltpu.async_copy(ut_hbm.at[:, pl.ds(j * W, W)], buf, sem)

        def wait_for(j, buf, sem):
            pltpu.make_async_copy(ut_hbm.at[:, pl.ds(j * W, W)], buf,
                                  sem).wait()

        def process(blk_ref, j, slot0):
            def rescan(ii, o2):
                uv = ul_v[pl.ds(ii * L, L)]
                bv = bl_v[pl.ds(ii * L, L)]
                jv = lax.shift_right_logical(uv, 7)
                m = (jv == j) & (ii * L + iota < nmatch)
                plsc.store_compressed(lj_v.at[pl.ds(o2, L)], uv & (W - 1), mask=m)
                plsc.store_compressed(bj_v.at[pl.ds(o2, L)], bv, mask=m)
                return o2 + jnp.max(plsc.all_reduce_population_count(m))

            o2 = lax.fori_loop(0, nmvr, rescan, 0)

            def emit(cc, carry):
                lv = lj_v[pl.ds(cc * L, L)] & (W - 1)
                bjv = bj_v[pl.ds(cc * L, L)]
                kvec = slot0 + cc * L + iota
                mk = (cc * L + iota) < o2
                plsc.store_scatter(bg_v, [kvec], bjv, mask=mk)
                for c in range(UD):
                    vals = plsc.load_gather(
                        blk_ref, [jnp.full((L,), c, jnp.int32), lv])
                    plsc.store_scatter(
                        stag_v, [kvec, jnp.full((L,), c, jnp.int32)], vals,
                        mask=mk)
                return carry

            nch = (o2 + (L - 1)) // L
            lax.fori_loop(0, nch, emit, 0)
            return slot0 + o2

        fire(lo, blka_v, sem_a)

        def pair(kk, slot0):
            j0 = lo + 2 * kk
            fire(j0 + 1, blkb_v, sem_b)
            wait_for(j0, blka_v, sem_a)
            slot1 = process(blka_v, j0, slot0)

            @pl.when(kk < 11)
            def _():
                fire(j0 + 2, blka_v, sem_a)

            wait_for(j0 + 1, blkb_v, sem_b)
            return process(blkb_v, j0 + 1, slot1)

        slot = lax.fori_loop(0, 12, pair, 0)

        @pl.when(lo + 24 < hi)
        def _():
            cp = pltpu.async_copy(
                ut_hbm.at[:, pl.ds((lo + 24) * W, W)], blka_v, sem_a)
            cp.wait()
            process(blka_v, lo + 24, slot)

        ocp = pltpu.async_copy(stag_v, outu_hbm.at[bg_v], sem_sc)
        ocp.wait()

    out_u, out_t = body(user_id, timestamp, ut, utail, ttz, buckets)
    return out_u[:B] + out_t


# speedup vs baseline: 3.6673x; 3.6673x over previous
import functools

import jax
import jax.numpy as jnp
from jax import lax
from jax.experimental import pallas as pl
from jax.experimental.pallas import tpu as pltpu
from jax.experimental.pallas import tpu_sc as plsc


def kernel(user_id, timestamp, user_table, time_table, buckets):
    B = user_id.shape[0]
    UD = user_table.shape[1]
    TD = time_table.shape[1]
    NB = buckets.shape[0]

    info = plsc.get_sparse_core_info()
    NC, NS, L = info.num_cores, info.num_subcores, info.num_lanes
    NW = NC * NS
    bpw = B // NW
    nq = 4
    qrows = bpw // nq
    qch = qrows // (2 * L)

    mesh = plsc.VectorSubcoreMesh(core_axis_name="c", subcore_axis_name="s")

    @functools.partial(
        pl.kernel,
        out_type=jax.ShapeDtypeStruct((2 * B, UD), jnp.float32),
        mesh=mesh,
        compiler_params=pltpu.CompilerParams(
            needs_layout_passes=False, use_tc_tiling_on_sc=False
        ),
        scratch_types=[
            pltpu.VMEM((bpw,), jnp.int32),
            pltpu.VMEM((bpw,), jnp.float32),
            pltpu.VMEM((NB,), jnp.float32),
            pltpu.VMEM((bpw,), jnp.int32),
            pltpu.VMEM((2 * bpw,), jnp.int32),
            pltpu.VMEM((2 * bpw, UD), jnp.float32),
            pltpu.SemaphoreType.DMA,
            pltpu.SemaphoreType.DMA,
            pltpu.SemaphoreType.DMA,
        ],
    )
    def body(uid_hbm, ts_hbm, utab_hbm, ttab_hbm, bkt_hbm, out_hbm,
             uidx_v, ts_v, bkt_v, tidx_v, srow_v, rows_v, sem_u, sem_t,
             sem_o):
        wid = lax.axis_index("s") * NC + lax.axis_index("c")
        base = wid * bpw
        iota = lax.iota(jnp.int32, L)

        pltpu.sync_copy(uid_hbm.at[pl.ds(base, bpw)], uidx_v)
        ucopy = pltpu.async_copy(utab_hbm.at[uidx_v], rows_v.at[pl.ds(0, bpw)],
                                 sem_u)

        pltpu.sync_copy(bkt_hbm, bkt_v)
        pltpu.sync_copy(ts_hbm.at[pl.ds(base, bpw)], ts_v)

        def search_pair(c, carry):
            v0 = ts_v[pl.ds(2 * c * L, L)]
            v1 = ts_v[pl.ds((2 * c + 1) * L, L)]
            a0 = jnp.zeros((L,), jnp.int32)
            a1 = jnp.zeros((L,), jnp.int32)
            k = NB
            while k >= 1:
                p0 = jnp.minimum(a0 + (k - 1), NB - 1)
                p1 = jnp.minimum(a1 + (k - 1), NB - 1)
                b0 = plsc.load_gather(bkt_v, [p0])
                b1 = plsc.load_gather(bkt_v, [p1])
                a0 = jnp.where((b0 <= v0) & (a0 + k <= NB), a0 + k, a0)
                a1 = jnp.where((b1 <= v1) & (a1 + k <= NB), a1 + k, a1)
                k //= 2
            tidx_v[pl.ds(2 * c * L, L)] = a0
            tidx_v[pl.ds((2 * c + 1) * L, L)] = a1
            s = (base + 2 * c * L) * 2 + iota * 2
            srow_v[pl.ds(2 * c * L, L)] = s
            srow_v[pl.ds((2 * c + 1) * L, L)] = s + 2 * L
            srow_v[pl.ds(bpw + 2 * c * L, L)] = s + 1
            srow_v[pl.ds(bpw + (2 * c + 1) * L, L)] = s + 2 * L + 1
            return carry

        tq = []
        for q in range(nq):
            lax.fori_loop(q * qch, (q + 1) * qch, search_pair, 0)
            tq.append(pltpu.async_copy(
                ttab_hbm.at[tidx_v.at[pl.ds(q * qrows, qrows)]],
                rows_v.at[pl.ds(bpw + q * qrows, qrows)], sem_t))

        ucopy.wait()
        for cp in tq:
            cp.wait()
        pltpu.async_copy(rows_v, out_hbm.at[srow_v], sem_o).wait()

    out2 = body(user_id, timestamp, user_table, time_table, buckets)
    return out2.reshape(B, UD + TD)


# speedup vs baseline: 3.6772x; 1.0027x over previous
import functools

import jax
import jax.numpy as jnp
from jax import lax
from jax.experimental import pallas as pl
from jax.experimental.pallas import tpu as pltpu
from jax.experimental.pallas import tpu_sc as plsc


def kernel(user_id, timestamp, user_table, time_table, buckets):
    B = user_id.shape[0]
    UD = user_table.shape[1]
    TD = time_table.shape[1]
    NB = buckets.shape[0]

    info = plsc.get_sparse_core_info()
    NC, NS, L = info.num_cores, info.num_subcores, info.num_lanes
    NW = NC * NS
    bpw = B // NW
    nq = 4
    qrows = bpw // nq
    qch = qrows // (2 * L)

    mesh = plsc.VectorSubcoreMesh(core_axis_name="c", subcore_axis_name="s")

    @functools.partial(
        pl.kernel,
        out_type=jax.ShapeDtypeStruct((2 * B, UD), jnp.float32),
        mesh=mesh,
        compiler_params=pltpu.CompilerParams(
            needs_layout_passes=False, use_tc_tiling_on_sc=False
        ),
        scratch_types=[
            pltpu.VMEM((bpw,), jnp.int32),
            pltpu.VMEM((bpw,), jnp.float32),
            pltpu.VMEM((NB,), jnp.float32),
            pltpu.VMEM((bpw,), jnp.int32),
            pltpu.VMEM((bpw,), jnp.int32),
            pltpu.VMEM((bpw,), jnp.int32),
            pltpu.VMEM((2 * bpw, UD), jnp.float32),
            pltpu.SemaphoreType.DMA,
            pltpu.SemaphoreType.DMA,
            pltpu.SemaphoreType.DMA,
        ],
    )
    def body(uid_hbm, ts_hbm, utab_hbm, ttab_hbm, bkt_hbm, out_hbm,
             uidx_v, ts_v, bkt_v, tidx_v, srowu_v, srowt_v, rows_v, sem_u,
             sem_t, sem_o):
        wid = lax.axis_index("s") * NC + lax.axis_index("c")
        base = wid * bpw
        iota = lax.iota(jnp.int32, L)

        pltpu.sync_copy(uid_hbm.at[pl.ds(base, bpw)], uidx_v)
        ucopy = pltpu.async_copy(utab_hbm.at[uidx_v], rows_v.at[pl.ds(0, bpw)],
                                 sem_u)

        pltpu.sync_copy(bkt_hbm, bkt_v)
        pltpu.sync_copy(ts_hbm.at[pl.ds(base, bpw)], ts_v)

        def search_pair(c, carry):
            v0 = ts_v[pl.ds(2 * c * L, L)]
            v1 = ts_v[pl.ds((2 * c + 1) * L, L)]
            a0 = jnp.zeros((L,), jnp.int32)
            a1 = jnp.zeros((L,), jnp.int32)
            k = NB
            while k >= 1:
                p0 = jnp.minimum(a0 + (k - 1), NB - 1)
                p1 = jnp.minimum(a1 + (k - 1), NB - 1)
                b0 = plsc.load_gather(bkt_v, [p0])
                b1 = plsc.load_gather(bkt_v, [p1])
                a0 = jnp.where((b0 <= v0) & (a0 + k <= NB), a0 + k, a0)
                a1 = jnp.where((b1 <= v1) & (a1 + k <= NB), a1 + k, a1)
                k //= 2
            tidx_v[pl.ds(2 * c * L, L)] = a0
            tidx_v[pl.ds((2 * c + 1) * L, L)] = a1
            s = (base + 2 * c * L) * 2 + iota * 2
            srowu_v[pl.ds(2 * c * L, L)] = s
            srowu_v[pl.ds((2 * c + 1) * L, L)] = s + 2 * L
            srowt_v[pl.ds(2 * c * L, L)] = s + 1
            srowt_v[pl.ds((2 * c + 1) * L, L)] = s + 2 * L + 1
            return carry

        tq = []
        for q in range(nq):
            lax.fori_loop(q * qch, (q + 1) * qch, search_pair, 0)
            tq.append(pltpu.async_copy(
                ttab_hbm.at[tidx_v.at[pl.ds(q * qrows, qrows)]],
                rows_v.at[pl.ds(bpw + q * qrows, qrows)], sem_t))

        ucopy.wait()
        oc_u = pltpu.async_copy(rows_v.at[pl.ds(0, bpw)],
                                out_hbm.at[srowu_v], sem_o)
        for cp in tq:
            cp.wait()
        oc_t = pltpu.async_copy(rows_v.at[pl.ds(bpw, bpw)],
                                out_hbm.at[srowt_v], sem_o)
        oc_u.wait()
        oc_t.wait()

    out2 = body(user_id, timestamp, user_table, time_table, buckets)
    return out2.reshape(B, UD + TD)
